# baseline (speedup 1.0000x reference)
import functools

import jax
import jax.numpy as jnp
from jax import lax
from jax.experimental import pallas as pl
from jax.experimental.pallas import tpu as pltpu
from jax.experimental.pallas import tpu_sc as plsc

BATCH = 16384
EMBED = 32
RBLK = 65536
PACK = 128 // EMBED
QROWS = RBLK // PACK
NC, NS = 2, 16
NW = NC * NS
BPW = BATCH // NW
CHUNK = 128
NCH = BPW // CHUNK


def _repack_body(tT_ref, out_ref):
  tb = tT_ref[...]
  tbstack = jnp.concatenate(
      [tb[:, q * QROWS:(q + 1) * QROWS] for q in range(PACK)], axis=0)
  eye = (lax.broadcasted_iota(jnp.int32, (128, 128), 0) ==
         lax.broadcasted_iota(jnp.int32, (128, 128), 1)).astype(jnp.float32)
  packed = lax.dot_general(tbstack, eye, (((0,), (0,)), ((), ())),
                           preferred_element_type=jnp.float32)
  out_ref[...] = packed.astype(jnp.bfloat16)


def _repack(tT, rows):
  nb = -(-rows // RBLK)
  return pl.pallas_call(
      _repack_body,
      grid=(nb,),
      in_specs=[pl.BlockSpec((EMBED, RBLK), lambda i: (0, i))],
      out_specs=pl.BlockSpec((QROWS, 128), lambda i: (i, 0)),
      out_shape=jax.ShapeDtypeStruct((nb * QROWS, 128), jnp.bfloat16),
      compiler_params=pltpu.CompilerParams(
          dimension_semantics=("arbitrary",)),
  )(tT)


def _gather_one_table(idx_hbm, tab_hbm, out_hbm, wid, idx_v, rows_v, sem):
  base = wid * BPW
  pltpu.sync_copy(idx_hbm.at[wid], idx_v)
  copies = []
  for j in range(NCH):
    dst = rows_v.at[pl.ds(j * CHUNK, CHUNK)]
    copies.append(pltpu.async_copy(tab_hbm.at[idx_v.at[j]], dst, sem))
  for c in copies:
    c.wait()
  pltpu.sync_copy(rows_v, out_hbm.at[pl.ds(base, BPW)])


def _gather_body(ulines_hbm, mlines_hbm, ut_hbm, mt_hbm, ue_out, me_out,
                 idx_v, rows_v, sem):
  wid = lax.axis_index("s") * NC + lax.axis_index("c")
  _gather_one_table(ulines_hbm, ut_hbm, ue_out, wid, idx_v, rows_v, sem)
  _gather_one_table(mlines_hbm, mt_hbm, me_out, wid, idx_v, rows_v, sem)


def _sc_gather(ulines, mlines, ut128, mt128):
  mesh = plsc.VectorSubcoreMesh(core_axis_name="c", subcore_axis_name="s")
  f = pl.kernel(
      _gather_body,
      out_type=[jax.ShapeDtypeStruct((BATCH, 128), jnp.bfloat16),
                jax.ShapeDtypeStruct((BATCH, 128), jnp.bfloat16)],
      mesh=mesh,
      scratch_types=[
          pltpu.VMEM((NCH, CHUNK), jnp.int32),
          pltpu.VMEM((BPW, 128), jnp.bfloat16),
          pltpu.SemaphoreType.DMA,
      ],
      compiler_params=pltpu.CompilerParams(use_tc_tiling_on_sc=False),
  )
  return f(ulines, mlines, ut128, mt128)


TILE = 2048


def _mlp_body(ue, me, usel, msel, w0a, w0b, b0, w1, b1, w2, b2, w3, b3, out):
  ue128 = ue[...].astype(jnp.float32)
  me128 = me[...].astype(jnp.float32)
  us = usel[...]
  ms = msel[...]
  xu = jnp.zeros((TILE, EMBED), jnp.float32)
  xm = jnp.zeros((TILE, EMBED), jnp.float32)
  for g in range(PACK):
    cu = ue128[:, g * EMBED:(g + 1) * EMBED]
    cm = me128[:, g * EMBED:(g + 1) * EMBED]
    xu = jnp.where(us == float(g), cu, xu)
    xm = jnp.where(ms == float(g), cm, xm)
  x = jnp.maximum(xu @ w0a[...] + xm @ w0b[...] + b0[...], 0.0)
  x = jnp.maximum(x @ w1[...] + b1[...], 0.0)
  x = jnp.maximum(x @ w2[...] + b2[...], 0.0)
  out[...] = x @ w3[...] + b3[...]


def _mlp(ue, me, usel, msel, W0, b0, W1, b1, W2, b2, W3, b3):
  full = lambda shape: pl.BlockSpec(shape, lambda i: (0, 0))
  return pl.pallas_call(
      _mlp_body,
      grid=(BATCH // TILE,),
      in_specs=[
          pl.BlockSpec((TILE, 128), lambda i: (i, 0)),
          pl.BlockSpec((TILE, 128), lambda i: (i, 0)),
          pl.BlockSpec((TILE, 1), lambda i: (i, 0)),
          pl.BlockSpec((TILE, 1), lambda i: (i, 0)),
          full((EMBED, 64)),
          full((EMBED, 64)),
          full((1, 64)),
          full((64, 32)),
          full((1, 32)),
          full((32, 16)),
          full((1, 16)),
          full((16, 1)),
          full((1, 1)),
      ],
      out_specs=pl.BlockSpec((TILE, 1), lambda i: (i, 0)),
      out_shape=jax.ShapeDtypeStruct((BATCH, 1), jnp.float32),
      compiler_params=pltpu.CompilerParams(
          dimension_semantics=("arbitrary",)),
  )(ue, me, usel, msel, W0[:EMBED], W0[EMBED:], b0.reshape(1, -1), W1,
    b1.reshape(1, -1), W2, b2.reshape(1, -1), W3, b3.reshape(1, -1))


def kernel(user, movie, user_table, movie_table, W0, b0, W1, b1, W2, b2, W3, b3):
  user = user.astype(jnp.int32)
  movie = movie.astype(jnp.int32)
  ut128 = _repack(user_table.T, 1000000)
  mt128 = _repack(movie_table.T, 100000)
  ul = (user // RBLK) * QROWS + (user % QROWS)
  ml = (movie // RBLK) * QROWS + (movie % QROWS)
  usel = ((user % RBLK) // QROWS).astype(jnp.float32).reshape(BATCH, 1)
  msel = ((movie % RBLK) // QROWS).astype(jnp.float32).reshape(BATCH, 1)
  ue, me = _sc_gather(ul.reshape(NW, NCH, CHUNK), ml.reshape(NW, NCH, CHUNK),
                      ut128, mt128)
  return _mlp(ue, me, usel, msel, W0, b0, W1, b1, W2, b2, W3, b3)


# speedup vs baseline: 2.3950x; 2.3950x over previous
import functools

import jax
import jax.numpy as jnp
from jax import lax
from jax.experimental import pallas as pl
from jax.experimental.pallas import tpu as pltpu
from jax.experimental.pallas import tpu_sc as plsc

BATCH = 16384
EMBED = 32
RBLK = 32768
PACK = 128 // EMBED
QROWS = RBLK // PACK
NC, NS = 2, 16
NW = NC * NS
BPW = BATCH // NW
CHUNK = 128
NCH = BPW // CHUNK


def _repack_body(tT_ref, out_ref):
  tb = tT_ref[...]
  tbstack = jnp.concatenate(
      [tb[:, q * QROWS:(q + 1) * QROWS] for q in range(PACK)], axis=0)
  eye = (lax.broadcasted_iota(jnp.int32, (128, 128), 0) ==
         lax.broadcasted_iota(jnp.int32, (128, 128), 1)).astype(jnp.float32)
  out_ref[...] = lax.dot_general(tbstack, eye, (((0,), (0,)), ((), ())),
                                 preferred_element_type=jnp.float32)


def _repack(tT, rows):
  nb = -(-rows // RBLK)
  return pl.pallas_call(
      _repack_body,
      grid=(nb,),
      in_specs=[pl.BlockSpec((EMBED, RBLK), lambda i: (0, i))],
      out_specs=pl.BlockSpec((QROWS, 128), lambda i: (i, 0)),
      out_shape=jax.ShapeDtypeStruct((nb * QROWS, 128), jnp.float32),
      compiler_params=pltpu.CompilerParams(
          dimension_semantics=("arbitrary",)),
  )(tT)


def _gather_one_table(idx_hbm, tab_hbm, out_hbm, wid, idx_v, rows_v, sem):
  base = wid * BPW
  pltpu.sync_copy(idx_hbm.at[wid], idx_v)
  copies = []
  for j in range(NCH):
    dst = rows_v.at[pl.ds(j * CHUNK, CHUNK)]
    copies.append(pltpu.async_copy(tab_hbm.at[idx_v.at[j]], dst, sem))
  for c in copies:
    c.wait()
  pltpu.sync_copy(rows_v, out_hbm.at[pl.ds(base, BPW)])


def _gather_body(ulines_hbm, mlines_hbm, ut_hbm, mt_hbm, ue_out, me_out,
                 idx_v, rows_v, sem):
  wid = lax.axis_index("s") * NC + lax.axis_index("c")
  _gather_one_table(ulines_hbm, ut_hbm, ue_out, wid, idx_v, rows_v, sem)
  _gather_one_table(mlines_hbm, mt_hbm, me_out, wid, idx_v, rows_v, sem)


def _sc_gather(ulines, mlines, ut128, mt128):
  mesh = plsc.VectorSubcoreMesh(core_axis_name="c", subcore_axis_name="s")
  f = pl.kernel(
      _gather_body,
      out_type=[jax.ShapeDtypeStruct((BATCH, 128), jnp.float32),
                jax.ShapeDtypeStruct((BATCH, 128), jnp.float32)],
      mesh=mesh,
      scratch_types=[
          pltpu.VMEM((NCH, CHUNK), jnp.int32),
          pltpu.VMEM((BPW, 128), jnp.float32),
          pltpu.SemaphoreType.DMA,
      ],
      compiler_params=pltpu.CompilerParams(use_tc_tiling_on_sc=False),
  )
  return f(ulines, mlines, ut128, mt128)


TILE = 2048


def _mlp_body(ue, me, usel, msel, w0a, w0b, b0, w1, b1, w2, b2, w3, b3, out):
  ue128 = ue[...]
  me128 = me[...]
  us = usel[...]
  ms = msel[...]
  xu = jnp.zeros((TILE, EMBED), jnp.float32)
  xm = jnp.zeros((TILE, EMBED), jnp.float32)
  for g in range(PACK):
    cu = ue128[:, g * EMBED:(g + 1) * EMBED]
    cm = me128[:, g * EMBED:(g + 1) * EMBED]
    xu = jnp.where(us == float(g), cu, xu)
    xm = jnp.where(ms == float(g), cm, xm)
  x = jnp.maximum(xu @ w0a[...] + xm @ w0b[...] + b0[...], 0.0)
  x = jnp.maximum(x @ w1[...] + b1[...], 0.0)
  x = jnp.maximum(x @ w2[...] + b2[...], 0.0)
  out[...] = x @ w3[...] + b3[...]


def _mlp(ue, me, usel, msel, W0, b0, W1, b1, W2, b2, W3, b3):
  full = lambda shape: pl.BlockSpec(shape, lambda i: (0, 0))
  return pl.pallas_call(
      _mlp_body,
      grid=(BATCH // TILE,),
      in_specs=[
          pl.BlockSpec((TILE, 128), lambda i: (i, 0)),
          pl.BlockSpec((TILE, 128), lambda i: (i, 0)),
          pl.BlockSpec((TILE, 1), lambda i: (i, 0)),
          pl.BlockSpec((TILE, 1), lambda i: (i, 0)),
          full((EMBED, 64)),
          full((EMBED, 64)),
          full((1, 64)),
          full((64, 32)),
          full((1, 32)),
          full((32, 16)),
          full((1, 16)),
          full((16, 1)),
          full((1, 1)),
      ],
      out_specs=pl.BlockSpec((TILE, 1), lambda i: (i, 0)),
      out_shape=jax.ShapeDtypeStruct((BATCH, 1), jnp.float32),
      compiler_params=pltpu.CompilerParams(
          dimension_semantics=("arbitrary",)),
  )(ue, me, usel, msel, W0[:EMBED], W0[EMBED:], b0.reshape(1, -1), W1,
    b1.reshape(1, -1), W2, b2.reshape(1, -1), W3, b3.reshape(1, -1))


def kernel(user, movie, user_table, movie_table, W0, b0, W1, b1, W2, b2, W3, b3):
  user = user.astype(jnp.int32)
  movie = movie.astype(jnp.int32)
  ut128 = _repack(user_table.T, 1000000)
  mt128 = _repack(movie_table.T, 100000)
  ul = (user // RBLK) * QROWS + (user % QROWS)
  ml = (movie // RBLK) * QROWS + (movie % QROWS)
  usel = ((user % RBLK) // QROWS).astype(jnp.float32).reshape(BATCH, 1)
  msel = ((movie % RBLK) // QROWS).astype(jnp.float32).reshape(BATCH, 1)
  ue, me = _sc_gather(ul.reshape(NW, NCH, CHUNK), ml.reshape(NW, NCH, CHUNK),
                      ut128, mt128)
  return _mlp(ue, me, usel, msel, W0, b0, W1, b1, W2, b2, W3, b3)
